# baseline (speedup 1.0000x reference)
import jax
import jax.numpy as jnp
from jax import lax
from jax.experimental import pallas as pl
from jax.experimental.pallas import tpu as pltpu
from jax.experimental.pallas import tpu_sc as plsc

NC = 2
NS = 16
NW = NC * NS
L = 16
K = 80

_SC_PARAMS = pltpu.CompilerParams(needs_layout_passes=False,
                                  use_tc_tiling_on_sc=False)


def _sc_degree(dst3, ew3):
    nb = dst3.shape[1]
    n = 10000

    def body(dst_hbm, ew_hbm, degp_out, deg_acc, dst_v, ew_v, z_v, stage_v):
        c = lax.axis_index("c")
        s = lax.axis_index("s")
        own = c * NS + s

        pltpu.sync_copy(dst_hbm.at[own], dst_v)
        pltpu.sync_copy(ew_hbm.at[own], ew_v)

        zeros16 = jnp.zeros((L,), jnp.float32)

        @pl.when(s == 0)
        def _():
            def zstep(r, carry):
                z_v[pl.ds(r * L, L)] = zeros16
                return carry
            lax.fori_loop(0, n // L, zstep, 0)
            pltpu.sync_copy(z_v, deg_acc)

        plsc.subcore_barrier()

        def dstep(j, carry):
            pltpu.sync_copy(ew_v.at[j], deg_acc.at[dst_v.at[j]], add=True)
            return carry
        lax.fori_loop(0, nb, dstep, 0)

        plsc.subcore_barrier()

        @pl.when(s == 0)
        def _():
            pltpu.sync_copy(deg_acc, stage_v)
            pltpu.sync_copy(stage_v, degp_out.at[c])

    mesh = plsc.VectorSubcoreMesh(core_axis_name="c", subcore_axis_name="s")
    return pl.kernel(
        body,
        out_type=jax.ShapeDtypeStruct((NC, n), jnp.float32),
        mesh=mesh,
        compiler_params=_SC_PARAMS,
        scratch_types=[
            pltpu.VMEM_SHARED((n,), jnp.float32),
            pltpu.VMEM((nb, K), jnp.int32),
            pltpu.VMEM((nb, K), jnp.float32),
            pltpu.VMEM((n,), jnp.float32),
            pltpu.VMEM((n,), jnp.float32),
        ],
    )(dst3, ew3)


def _mm_body(emb_ref, w_ref, degp_ref, xs_ref, dinv_ref):
    deg = degp_ref[0, :, :] + degp_ref[1, :, :] + 1.0
    dinv = lax.rsqrt(deg)
    x = jnp.dot(emb_ref[...], w_ref[...], preferred_element_type=jnp.float32)
    xs_ref[...] = dinv * x
    dinv_ref[...] = dinv


def _matmul_scale(emb, W, degp):
    n, d = emb.shape
    bn = 1000
    return pl.pallas_call(
        _mm_body,
        grid=(n // bn,),
        in_specs=[pl.BlockSpec((bn, d), lambda i: (i, 0)),
                  pl.BlockSpec((d, d), lambda i: (0, 0)),
                  pl.BlockSpec((NC, bn, 1), lambda i: (0, i, 0))],
        out_specs=[pl.BlockSpec((bn, d), lambda i: (i, 0)),
                   pl.BlockSpec((bn, 1), lambda i: (i, 0))],
        out_shape=[jax.ShapeDtypeStruct((n, d), jnp.float32),
                   jax.ShapeDtypeStruct((n, 1), jnp.float32)],
    )(emb, W, degp.reshape(NC, n, 1))


def _sc_message_pass(xs, src3, dst3, ew3):
    n, d = xs.shape
    nb = src3.shape[1]
    n_chunks = n // K
    chunk_rounds = -(-n_chunks // NS)

    def body(xs_hbm, src_hbm, dst_hbm, ew_hbm, m_out,
             m_acc, src_v, dst_v, w_v, rows, gsem, ssem):
        c = lax.axis_index("c")
        s = lax.axis_index("s")
        own = c * NS + s

        pltpu.sync_copy(src_hbm.at[own], src_v)
        pltpu.sync_copy(dst_hbm.at[own], dst_v)
        pltpu.sync_copy(ew_hbm.at[own], w_v)

        zeros16 = jnp.zeros((L,), jnp.float32)

        def zrow(r, carry):
            for u in range(d // L):
                rows[0, r, pl.ds(u * L, L)] = zeros16
            return carry
        lax.fori_loop(0, K, zrow, 0)

        for i in range(chunk_rounds):
            j = i * NS + s

            @pl.when(j < n_chunks)
            def _(j=j):
                pltpu.sync_copy(rows.at[0], m_acc.at[pl.ds(j * K, K)])

        plsc.subcore_barrier()

        def fire_gather(j, b):
            pltpu.async_copy(xs_hbm.at[src_v.at[j]], rows.at[b],
                             gsem.at[b])

        def do_batch(j, b):
            q = 1 - b
            pltpu.make_async_copy(xs_hbm.at[src_v.at[j]], rows.at[b],
                                  gsem.at[b]).wait()

            @plsc.parallel_loop(0, K, unroll=4)
            def _(r):
                wv = plsc.load_gather(
                    w_v, [jnp.full((L,), j, jnp.int32),
                          jnp.full((L,), r, jnp.int32)])
                for u in range(d // L):
                    rows[b, r, pl.ds(u * L, L)] = (
                        rows[b, r, pl.ds(u * L, L)] * wv)
            pltpu.async_copy(rows.at[b], m_acc.at[dst_v.at[j]],
                             ssem.at[b], add=True)

            @pl.when(j >= 1)
            def _():
                pltpu.make_async_copy(rows.at[q], m_acc.at[dst_v.at[j - 1]],
                                      ssem.at[q]).wait()

            @pl.when(j + 1 < nb)
            def _():
                fire_gather(j + 1, q)

        fire_gather(0, 0)
        do_batch(0, 0)

        def mstep(t, carry):
            do_batch(2 * t + 1, 1)
            do_batch(2 * t + 2, 0)
            return carry
        lax.fori_loop(0, (nb - 1) // 2, mstep, 0)
        pltpu.make_async_copy(rows.at[0], m_acc.at[dst_v.at[nb - 1]],
                              ssem.at[0]).wait()

        plsc.subcore_barrier()

        out_base = c * n
        for i in range(chunk_rounds):
            j = i * NS + s

            @pl.when(j < n_chunks)
            def _(j=j):
                pltpu.sync_copy(m_acc.at[pl.ds(j * K, K)], rows.at[0])
                pltpu.sync_copy(rows.at[0],
                                m_out.at[pl.ds(out_base + j * K, K)])

    mesh = plsc.VectorSubcoreMesh(core_axis_name="c", subcore_axis_name="s")
    return pl.kernel(
        body,
        out_type=jax.ShapeDtypeStruct((NC * n, d), jnp.float32),
        mesh=mesh,
        compiler_params=_SC_PARAMS,
        scratch_types=[
            pltpu.VMEM_SHARED((n, d), jnp.float32),
            pltpu.VMEM((nb, K), jnp.int32),
            pltpu.VMEM((nb, K), jnp.int32),
            pltpu.VMEM((nb, K), jnp.float32),
            pltpu.VMEM((2, K, d), jnp.float32),
            pltpu.SemaphoreType.DMA((2,)),
            pltpu.SemaphoreType.DMA((2,)),
        ],
    )(xs, src3, dst3, ew3)


def _fin_body(m0_ref, m1_ref, xs_ref, dinv_ref, b_ref, o_ref):
    dinv = dinv_ref[...]
    o_ref[...] = dinv * (m0_ref[...] + m1_ref[...] + xs_ref[...]) + b_ref[...]


def _finalize(m_parts, xs, dinv, b):
    n, d = xs.shape
    bn = 1000
    nblk = n // bn
    return pl.pallas_call(
        _fin_body,
        grid=(nblk,),
        in_specs=[
            pl.BlockSpec((bn, d), lambda i: (i, 0)),
            pl.BlockSpec((bn, d), lambda i: (i + nblk, 0)),
            pl.BlockSpec((bn, d), lambda i: (i, 0)),
            pl.BlockSpec((bn, 1), lambda i: (i, 0)),
            pl.BlockSpec((1, d), lambda i: (0, 0)),
        ],
        out_specs=pl.BlockSpec((bn, d), lambda i: (i, 0)),
        out_shape=jax.ShapeDtypeStruct((n, d), jnp.float32),
    )(m_parts, m_parts, xs, dinv, b.reshape(1, d))


def kernel(emb, W, b, edge_index, edge_weight):
    n, d = emb.shape
    e = edge_weight.shape[0]
    nb = e // (NW * K)
    src3 = edge_index[0].astype(jnp.int32).reshape(NW, nb, K)
    dst3 = edge_index[1].astype(jnp.int32).reshape(NW, nb, K)
    ew3 = edge_weight.astype(jnp.float32).reshape(NW, nb, K)
    degp = _sc_degree(dst3, ew3)
    xs, dinv = _matmul_scale(emb, W, degp)
    m_parts = _sc_message_pass(xs, src3, dst3, ew3)
    return _finalize(m_parts, xs, dinv, b)


# speedup vs baseline: 1.2041x; 1.2041x over previous
import jax
import jax.numpy as jnp
from jax import lax
from jax.experimental import pallas as pl
from jax.experimental.pallas import tpu as pltpu
from jax.experimental.pallas import tpu_sc as plsc

NC = 2
NS = 16
NW = NC * NS
L = 16
K = 80

_SC_PARAMS = pltpu.CompilerParams(needs_layout_passes=False,
                                  use_tc_tiling_on_sc=False)


def _sc_degree(dst3, ew3):
    nb = dst3.shape[1]
    n = 10000

    def body(dst_hbm, ew_hbm, degp_out, deg_acc, dst_v, ew_v, z_v, stage_v):
        c = lax.axis_index("c")
        s = lax.axis_index("s")
        own = c * NS + s

        pltpu.sync_copy(dst_hbm.at[own], dst_v)
        pltpu.sync_copy(ew_hbm.at[own], ew_v)

        zeros16 = jnp.zeros((L,), jnp.float32)

        @pl.when(s == 0)
        def _():
            def zstep(r, carry):
                z_v[pl.ds(r * L, L)] = zeros16
                return carry
            lax.fori_loop(0, n // L, zstep, 0)
            pltpu.sync_copy(z_v, deg_acc)

        plsc.subcore_barrier()

        def dstep(j, carry):
            pltpu.sync_copy(ew_v.at[j], deg_acc.at[dst_v.at[j]], add=True)
            return carry
        lax.fori_loop(0, nb, dstep, 0)

        plsc.subcore_barrier()

        @pl.when(s == 0)
        def _():
            pltpu.sync_copy(deg_acc, stage_v)
            pltpu.sync_copy(stage_v, degp_out.at[c])

    mesh = plsc.VectorSubcoreMesh(core_axis_name="c", subcore_axis_name="s")
    return pl.kernel(
        body,
        out_type=jax.ShapeDtypeStruct((NC, n), jnp.float32),
        mesh=mesh,
        compiler_params=_SC_PARAMS,
        scratch_types=[
            pltpu.VMEM_SHARED((n,), jnp.float32),
            pltpu.VMEM((nb, K), jnp.int32),
            pltpu.VMEM((nb, K), jnp.float32),
            pltpu.VMEM((n,), jnp.float32),
            pltpu.VMEM((n,), jnp.float32),
        ],
    )(dst3, ew3)


def _mm_body(emb_ref, w_ref, degp_ref, xs_ref, dinv_ref):
    deg = degp_ref[0, :, :] + degp_ref[1, :, :] + 1.0
    dinv = lax.rsqrt(deg)
    x = jnp.dot(emb_ref[...], w_ref[...], preferred_element_type=jnp.float32)
    xs_ref[...] = dinv * x
    dinv_ref[...] = dinv


def _matmul_scale(emb, W, degp):
    n, d = emb.shape
    bn = 1000
    return pl.pallas_call(
        _mm_body,
        grid=(n // bn,),
        in_specs=[pl.BlockSpec((bn, d), lambda i: (i, 0)),
                  pl.BlockSpec((d, d), lambda i: (0, 0)),
                  pl.BlockSpec((NC, bn, 1), lambda i: (0, i, 0))],
        out_specs=[pl.BlockSpec((bn, d), lambda i: (i, 0)),
                   pl.BlockSpec((bn, 1), lambda i: (i, 0))],
        out_shape=[jax.ShapeDtypeStruct((n, d), jnp.float32),
                   jax.ShapeDtypeStruct((n, 1), jnp.float32)],
    )(emb, W, degp.reshape(NC, n, 1))


def _sc_message_pass(xs, src3, dst3, ew3):
    n, d = xs.shape
    nb = src3.shape[1]
    n_chunks = n // K
    chunk_rounds = -(-n_chunks // NS)

    def body(xs_hbm, src_hbm, dst_hbm, ew_hbm, m_out,
             m_acc, src_v, dst_v, w_v, rows, gsem, ssem):
        c = lax.axis_index("c")
        s = lax.axis_index("s")
        own = c * NS + s

        pltpu.sync_copy(src_hbm.at[own], src_v)
        pltpu.sync_copy(dst_hbm.at[own], dst_v)
        pltpu.sync_copy(ew_hbm.at[own], w_v)

        zeros16 = jnp.zeros((L,), jnp.float32)

        def zrow(r, carry):
            for u in range(d // L):
                rows[0, r, pl.ds(u * L, L)] = zeros16
            return carry
        lax.fori_loop(0, K, zrow, 0)

        for i in range(chunk_rounds):
            j = i * NS + s

            @pl.when(j < n_chunks)
            def _(j=j):
                pltpu.sync_copy(rows.at[0], m_acc.at[pl.ds(j * K, K)])

        plsc.subcore_barrier()

        def fire_gather(j, b):
            pltpu.async_copy(xs_hbm.at[src_v.at[j]], rows.at[b],
                             gsem.at[b])

        def do_batch(j, b):
            q = 1 - b
            pltpu.make_async_copy(xs_hbm.at[src_v.at[j]], rows.at[b],
                                  gsem.at[b]).wait()

            @pl.when(j >= 1)
            def _():
                pltpu.make_async_copy(rows.at[q], m_acc.at[dst_v.at[j - 1]],
                                      ssem.at[q]).wait()

            @pl.when(j + 1 < nb)
            def _():
                fire_gather(j + 1, q)

            @plsc.parallel_loop(0, K, unroll=4)
            def _(r):
                wv = plsc.load_gather(
                    w_v, [jnp.full((L,), j, jnp.int32),
                          jnp.full((L,), r, jnp.int32)])
                for u in range(d // L):
                    rows[b, r, pl.ds(u * L, L)] = (
                        rows[b, r, pl.ds(u * L, L)] * wv)
            pltpu.async_copy(rows.at[b], m_acc.at[dst_v.at[j]],
                             ssem.at[b], add=True)

        fire_gather(0, 0)
        do_batch(0, 0)

        def mstep(t, carry):
            do_batch(2 * t + 1, 1)
            do_batch(2 * t + 2, 0)
            return carry
        lax.fori_loop(0, (nb - 1) // 2, mstep, 0)
        pltpu.make_async_copy(rows.at[0], m_acc.at[dst_v.at[nb - 1]],
                              ssem.at[0]).wait()

        plsc.subcore_barrier()

        out_base = c * n
        for i in range(chunk_rounds):
            j = i * NS + s

            @pl.when(j < n_chunks)
            def _(j=j):
                pltpu.sync_copy(m_acc.at[pl.ds(j * K, K)], rows.at[0])
                pltpu.sync_copy(rows.at[0],
                                m_out.at[pl.ds(out_base + j * K, K)])

    mesh = plsc.VectorSubcoreMesh(core_axis_name="c", subcore_axis_name="s")
    return pl.kernel(
        body,
        out_type=jax.ShapeDtypeStruct((NC * n, d), jnp.float32),
        mesh=mesh,
        compiler_params=_SC_PARAMS,
        scratch_types=[
            pltpu.VMEM_SHARED((n, d), jnp.float32),
            pltpu.VMEM((nb, K), jnp.int32),
            pltpu.VMEM((nb, K), jnp.int32),
            pltpu.VMEM((nb, K), jnp.float32),
            pltpu.VMEM((2, K, d), jnp.float32),
            pltpu.SemaphoreType.DMA((2,)),
            pltpu.SemaphoreType.DMA((2,)),
        ],
    )(xs, src3, dst3, ew3)


def _fin_body(m0_ref, m1_ref, xs_ref, dinv_ref, b_ref, o_ref):
    dinv = dinv_ref[...]
    o_ref[...] = dinv * (m0_ref[...] + m1_ref[...] + xs_ref[...]) + b_ref[...]


def _finalize(m_parts, xs, dinv, b):
    n, d = xs.shape
    bn = 1000
    nblk = n // bn
    return pl.pallas_call(
        _fin_body,
        grid=(nblk,),
        in_specs=[
            pl.BlockSpec((bn, d), lambda i: (i, 0)),
            pl.BlockSpec((bn, d), lambda i: (i + nblk, 0)),
            pl.BlockSpec((bn, d), lambda i: (i, 0)),
            pl.BlockSpec((bn, 1), lambda i: (i, 0)),
            pl.BlockSpec((1, d), lambda i: (0, 0)),
        ],
        out_specs=pl.BlockSpec((bn, d), lambda i: (i, 0)),
        out_shape=jax.ShapeDtypeStruct((n, d), jnp.float32),
    )(m_parts, m_parts, xs, dinv, b.reshape(1, d))


def kernel(emb, W, b, edge_index, edge_weight):
    n, d = emb.shape
    e = edge_weight.shape[0]
    nb = e // (NW * K)
    src3 = edge_index[0].astype(jnp.int32).reshape(NW, nb, K)
    dst3 = edge_index[1].astype(jnp.int32).reshape(NW, nb, K)
    ew3 = edge_weight.astype(jnp.float32).reshape(NW, nb, K)
    degp = _sc_degree(dst3, ew3)
    xs, dinv = _matmul_scale(emb, W, degp)
    m_parts = _sc_message_pass(xs, src3, dst3, ew3)
    return _finalize(m_parts, xs, dinv, b)


# speedup vs baseline: 1.2423x; 1.0317x over previous
import jax
import jax.numpy as jnp
from jax import lax
from jax.experimental import pallas as pl
from jax.experimental.pallas import tpu as pltpu
from jax.experimental.pallas import tpu_sc as plsc

NC = 2
NS = 16
NW = NC * NS
L = 16
K = 80

_SC_PARAMS = pltpu.CompilerParams(needs_layout_passes=False,
                                  use_tc_tiling_on_sc=False)


def _sc_degree(dst3, ew3):
    nb = dst3.shape[1]
    n = 10000

    def body(dst_hbm, ew_hbm, degp_out, deg_acc, dst_v, ew_v, z_v, stage_v,
             dsem):
        c = lax.axis_index("c")
        s = lax.axis_index("s")
        own = c * NS + s

        pltpu.sync_copy(dst_hbm.at[own], dst_v)
        pltpu.sync_copy(ew_hbm.at[own], ew_v)

        zeros16 = jnp.zeros((L,), jnp.float32)

        @pl.when(s == 0)
        def _():
            def zstep(r, carry):
                z_v[pl.ds(r * L, L)] = zeros16
                return carry
            lax.fori_loop(0, n // L, zstep, 0)
            pltpu.sync_copy(z_v, deg_acc)

        plsc.subcore_barrier()

        for k in range(5):
            pltpu.async_copy(ew_v.at[k], deg_acc.at[dst_v.at[k]],
                             dsem.at[k], add=True)

        def dstep(t, carry):
            for k in range(5):
                j = t * 5 + k
                pltpu.make_async_copy(ew_v.at[j], deg_acc.at[dst_v.at[j]],
                                      dsem.at[k]).wait()

                @pl.when(t + 1 < nb // 5)
                def _(j=j, k=k):
                    pltpu.async_copy(ew_v.at[j + 5],
                                     deg_acc.at[dst_v.at[j + 5]],
                                     dsem.at[k], add=True)
            return carry
        lax.fori_loop(0, nb // 5, dstep, 0)

        plsc.subcore_barrier()

        @pl.when(s == 0)
        def _():
            pltpu.sync_copy(deg_acc, stage_v)
            pltpu.sync_copy(stage_v, degp_out.at[c])

    mesh = plsc.VectorSubcoreMesh(core_axis_name="c", subcore_axis_name="s")
    return pl.kernel(
        body,
        out_type=jax.ShapeDtypeStruct((NC, n), jnp.float32),
        mesh=mesh,
        compiler_params=_SC_PARAMS,
        scratch_types=[
            pltpu.VMEM_SHARED((n,), jnp.float32),
            pltpu.VMEM((nb, K), jnp.int32),
            pltpu.VMEM((nb, K), jnp.float32),
            pltpu.VMEM((n,), jnp.float32),
            pltpu.VMEM((n,), jnp.float32),
            pltpu.SemaphoreType.DMA((5,)),
        ],
    )(dst3, ew3)


def _mm_body(emb_ref, w_ref, degp_ref, xs_ref, dinv_ref):
    deg = degp_ref[0, :, :] + degp_ref[1, :, :] + 1.0
    dinv = lax.rsqrt(deg)
    x = jnp.dot(emb_ref[...], w_ref[...], preferred_element_type=jnp.float32)
    xs_ref[...] = dinv * x
    dinv_ref[...] = dinv


def _matmul_scale(emb, W, degp):
    n, d = emb.shape
    bn = 1000
    return pl.pallas_call(
        _mm_body,
        grid=(n // bn,),
        in_specs=[pl.BlockSpec((bn, d), lambda i: (i, 0)),
                  pl.BlockSpec((d, d), lambda i: (0, 0)),
                  pl.BlockSpec((NC, bn, 1), lambda i: (0, i, 0))],
        out_specs=[pl.BlockSpec((bn, d), lambda i: (i, 0)),
                   pl.BlockSpec((bn, 1), lambda i: (i, 0))],
        out_shape=[jax.ShapeDtypeStruct((n, d), jnp.float32),
                   jax.ShapeDtypeStruct((n, 1), jnp.float32)],
    )(emb, W, degp.reshape(NC, n, 1))


def _sc_message_pass(xs, src3, dst3, ew3):
    n, d = xs.shape
    nb = src3.shape[1]
    n_chunks = n // K
    chunk_rounds = -(-n_chunks // NS)

    def body(xs_hbm, src_hbm, dst_hbm, ew_hbm, m_out,
             m_acc, src_v, dst_v, w_v, rows, gsem, ssem):
        c = lax.axis_index("c")
        s = lax.axis_index("s")
        own = c * NS + s

        pltpu.sync_copy(src_hbm.at[own], src_v)
        pltpu.sync_copy(dst_hbm.at[own], dst_v)
        pltpu.sync_copy(ew_hbm.at[own], w_v)

        zeros16 = jnp.zeros((L,), jnp.float32)

        def zrow(r, carry):
            for u in range(d // L):
                rows[0, r, pl.ds(u * L, L)] = zeros16
            return carry
        lax.fori_loop(0, K, zrow, 0)

        for i in range(chunk_rounds):
            j = i * NS + s

            @pl.when(j < n_chunks)
            def _(j=j):
                pltpu.sync_copy(rows.at[0], m_acc.at[pl.ds(j * K, K)])

        plsc.subcore_barrier()

        def fire_gather(j, b):
            pltpu.async_copy(xs_hbm.at[src_v.at[j]], rows.at[b],
                             gsem.at[b])

        def do_batch(j, b):
            q = 1 - b
            pltpu.make_async_copy(xs_hbm.at[src_v.at[j]], rows.at[b],
                                  gsem.at[b]).wait()

            @pl.when(j >= 1)
            def _():
                pltpu.make_async_copy(rows.at[q], m_acc.at[dst_v.at[j - 1]],
                                      ssem.at[q]).wait()

            @pl.when(j + 1 < nb)
            def _():
                fire_gather(j + 1, q)

            @plsc.parallel_loop(0, K, unroll=4)
            def _(r):
                wv = plsc.load_gather(
                    w_v, [jnp.full((L,), j, jnp.int32),
                          jnp.full((L,), r, jnp.int32)])
                for u in range(d // L):
                    rows[b, r, pl.ds(u * L, L)] = (
                        rows[b, r, pl.ds(u * L, L)] * wv)
            pltpu.async_copy(rows.at[b], m_acc.at[dst_v.at[j]],
                             ssem.at[b], add=True)

        fire_gather(0, 0)
        do_batch(0, 0)

        def mstep(t, carry):
            do_batch(2 * t + 1, 1)
            do_batch(2 * t + 2, 0)
            return carry
        lax.fori_loop(0, (nb - 1) // 2, mstep, 0)
        pltpu.make_async_copy(rows.at[0], m_acc.at[dst_v.at[nb - 1]],
                              ssem.at[0]).wait()

        plsc.subcore_barrier()

        out_base = c * n
        for i in range(chunk_rounds):
            j = i * NS + s

            @pl.when(j < n_chunks)
            def _(j=j):
                pltpu.sync_copy(m_acc.at[pl.ds(j * K, K)], rows.at[0])
                pltpu.sync_copy(rows.at[0],
                                m_out.at[pl.ds(out_base + j * K, K)])

    mesh = plsc.VectorSubcoreMesh(core_axis_name="c", subcore_axis_name="s")
    return pl.kernel(
        body,
        out_type=jax.ShapeDtypeStruct((NC * n, d), jnp.float32),
        mesh=mesh,
        compiler_params=_SC_PARAMS,
        scratch_types=[
            pltpu.VMEM_SHARED((n, d), jnp.float32),
            pltpu.VMEM((nb, K), jnp.int32),
            pltpu.VMEM((nb, K), jnp.int32),
            pltpu.VMEM((nb, K), jnp.float32),
            pltpu.VMEM((2, K, d), jnp.float32),
            pltpu.SemaphoreType.DMA((2,)),
            pltpu.SemaphoreType.DMA((2,)),
        ],
    )(xs, src3, dst3, ew3)


def _fin_body(m0_ref, m1_ref, xs_ref, dinv_ref, b_ref, o_ref):
    dinv = dinv_ref[...]
    o_ref[...] = dinv * (m0_ref[...] + m1_ref[...] + xs_ref[...]) + b_ref[...]


def _finalize(m_parts, xs, dinv, b):
    n, d = xs.shape
    bn = 1000
    nblk = n // bn
    return pl.pallas_call(
        _fin_body,
        grid=(nblk,),
        in_specs=[
            pl.BlockSpec((bn, d), lambda i: (i, 0)),
            pl.BlockSpec((bn, d), lambda i: (i + nblk, 0)),
            pl.BlockSpec((bn, d), lambda i: (i, 0)),
            pl.BlockSpec((bn, 1), lambda i: (i, 0)),
            pl.BlockSpec((1, d), lambda i: (0, 0)),
        ],
        out_specs=pl.BlockSpec((bn, d), lambda i: (i, 0)),
        out_shape=jax.ShapeDtypeStruct((n, d), jnp.float32),
    )(m_parts, m_parts, xs, dinv, b.reshape(1, d))


def kernel(emb, W, b, edge_index, edge_weight):
    n, d = emb.shape
    e = edge_weight.shape[0]
    nb = e // (NW * K)
    src3 = edge_index[0].astype(jnp.int32).reshape(NW, nb, K)
    dst3 = edge_index[1].astype(jnp.int32).reshape(NW, nb, K)
    ew3 = edge_weight.astype(jnp.float32).reshape(NW, nb, K)
    degp = _sc_degree(dst3, ew3)
    xs, dinv = _matmul_scale(emb, W, degp)
    m_parts = _sc_message_pass(xs, src3, dst3, ew3)
    return _finalize(m_parts, xs, dinv, b)
